# Initial kernel scaffold; baseline (speedup 1.0000x reference)
import functools

import jax
import jax.numpy as jnp
from jax import lax
from jax.experimental import pallas as pl
from jax.experimental.pallas import tpu as pltpu
from jax.experimental.pallas import tpu_sc as plsc

_V = 100000
_D = 128
_B = 4096 * 50
_NC = 2
_NS = 16
_NW = _NC * _NS
_RPW = _B // _NW
_CH = 128
_NCH = _RPW // _CH

_mesh = plsc.VectorSubcoreMesh(
    core_axis_name="c", subcore_axis_name="s", num_cores=_NC, num_subcores=_NS
)


def _tile_body(x_hbm, w_hbm, sz_hbm, out_hbm, idx_v, w_v, sz_v, out_v,
               sem_w, sem_sz):
    wid = lax.axis_index("s") * _NC + lax.axis_index("c")
    base0 = wid * _RPW
    lane = lax.iota(jnp.int32, 16)
    ge8 = lane // 8
    col_idx = [[64 * h + 4 * lane + j for j in range(4)] for h in range(2)]

    def chunk_body(t, carry):
        base = base0 + t * _CH
        pltpu.sync_copy(x_hbm.at[pl.ds(base, _CH)], idx_v)
        cw = pltpu.async_copy(w_hbm.at[idx_v], w_v, sem_w)
        cs = pltpu.async_copy(sz_hbm.at[idx_v], sz_v, sem_sz)
        cw.wait()
        cs.wait()

        def row_body(r, carry2):
            szr = sz_v[r]
            svec = plsc.bitcast(szr, jnp.float32)
            zvec = szr.astype(jnp.float32)
            rbase = jnp.full((16,), r * _D, jnp.int32)
            for h in range(2):
                wv = w_v[r, pl.ds(16 * h, 16)]
                sp = jnp.take(svec, ge8 + 2 * h, mode="promise_in_bounds")
                zp = jnp.take(zvec, ge8 + 2 * h + 4, mode="promise_in_bounds")
                for j in range(4):
                    b = (wv << (24 - 8 * j)) >> 24
                    f = (b.astype(jnp.float32) - zp) * sp
                    plsc.store_scatter(out_v, [rbase + col_idx[h][j]], f)
            return carry2

        lax.fori_loop(0, _CH, row_body, 0, unroll=2)
        pltpu.sync_copy(out_v, out_hbm.at[pl.ds(base * _D, _CH * _D)])
        return carry

    lax.fori_loop(0, _NCH, chunk_body, 0)


_lookup = pl.kernel(
    _tile_body,
    out_type=jax.ShapeDtypeStruct((_B * _D,), jnp.float32),
    mesh=_mesh,
    scratch_types=[
        pltpu.VMEM((_CH,), jnp.int32),
        pltpu.VMEM((_CH, 32), jnp.int32),
        pltpu.VMEM((_CH, 16), jnp.int32),
        pltpu.VMEM((_CH * _D,), jnp.float32),
        pltpu.SemaphoreType.DMA,
        pltpu.SemaphoreType.DMA,
    ],
)


@jax.jit
def kernel(x, weight, scale, zero_point):
    xf = x.reshape(-1)
    w32 = lax.bitcast_convert_type(weight.reshape(_V, _D // 4, 4), jnp.int32)
    sbits = lax.bitcast_convert_type(scale, jnp.int32)
    sz = jnp.concatenate(
        [sbits, zero_point, jnp.zeros((_V, 8), jnp.int32)], axis=1
    )
    out = _lookup(xf, w32, sz)
    return out.reshape(x.shape[0], x.shape[1], _D)


# speedup vs baseline: 1.4221x; 1.4221x over previous
import functools

import jax
import jax.numpy as jnp
from jax import lax
from jax.experimental import pallas as pl
from jax.experimental.pallas import tpu as pltpu
from jax.experimental.pallas import tpu_sc as plsc

_V = 100000
_D = 128
_B = 4096 * 50
_NC = 2
_NS = 16
_NW = _NC * _NS
_RPW = _B // _NW
_CH = 128
_NCH = _RPW // _CH

_mesh = plsc.VectorSubcoreMesh(
    core_axis_name="c", subcore_axis_name="s", num_cores=_NC, num_subcores=_NS
)


def _dyn_gather(v, idx):
    return lax.gather(
        v, idx[:, None],
        lax.GatherDimensionNumbers(
            offset_dims=(), collapsed_slice_dims=(0,), start_index_map=(0,)),
        (1,),
        mode=lax.GatherScatterMode.PROMISE_IN_BOUNDS)

def _tile_body(x_hbm, w_hbm, sz_hbm, out_hbm, idx_v, w_v, sz_v, out_v,
               sem_w, sem_sz):
    wid = lax.axis_index("s") * _NC + lax.axis_index("c")
    base0 = wid * _RPW
    lane = lax.iota(jnp.int32, 16)
    ge8 = lane // 8
    col_idx = [[64 * h + 4 * lane + j for j in range(4)] for h in range(2)]

    def chunk_body(t, carry):
        base = base0 + t * _CH
        pltpu.sync_copy(x_hbm.at[pl.ds(base, _CH)], idx_v)
        cw = pltpu.async_copy(w_hbm.at[idx_v], w_v, sem_w)
        cs = pltpu.async_copy(sz_hbm.at[idx_v], sz_v, sem_sz)
        cw.wait()
        cs.wait()

        def row_body(r, carry2):
            szr = sz_v[r]
            rbase = jnp.full((16,), r * _D, jnp.int32)
            for h in range(2):
                wv = w_v[r, pl.ds(16 * h, 16)]
                sp = _dyn_gather(szr, ge8 + 2 * h)
                zp = _dyn_gather(szr, ge8 + 2 * h + 4)
                for j in range(4):
                    b = (wv << (24 - 8 * j)) >> 24
                    f = (b.astype(jnp.float32) - zp) * sp
                    plsc.store_scatter(out_v, [rbase + col_idx[h][j]], f)
            return carry2

        lax.fori_loop(0, _CH, row_body, 0, unroll=2)
        pltpu.sync_copy(out_v, out_hbm.at[pl.ds(base * _D, _CH * _D)])
        return carry

    lax.fori_loop(0, _NCH, chunk_body, 0)


_lookup = pl.kernel(
    _tile_body,
    out_type=jax.ShapeDtypeStruct((_B * _D,), jnp.float32),
    mesh=_mesh,
    scratch_types=[
        pltpu.VMEM((_CH,), jnp.int32),
        pltpu.VMEM((_CH, 32), jnp.int32),
        pltpu.VMEM((_CH, 16), jnp.float32),
        pltpu.VMEM((_CH * _D,), jnp.float32),
        pltpu.SemaphoreType.DMA,
        pltpu.SemaphoreType.DMA,
    ],
    compiler_params=pltpu.CompilerParams(
        needs_layout_passes=False, use_tc_tiling_on_sc=False
    ),
)


@jax.jit
def kernel(x, weight, scale, zero_point):
    xf = x.reshape(-1)
    w32 = lax.bitcast_convert_type(weight.reshape(_V, _D // 4, 4), jnp.int32)
    sz = jnp.concatenate(
        [scale, zero_point.astype(jnp.float32), jnp.zeros((_V, 8), jnp.float32)],
        axis=1,
    )
    out = _lookup(xf, w32, sz)
    return out.reshape(x.shape[0], x.shape[1], _D)


# speedup vs baseline: 3.2731x; 2.3017x over previous
import jax
import jax.numpy as jnp
from jax import lax
from jax.experimental import pallas as pl
from jax.experimental.pallas import tpu as pltpu
from jax.experimental.pallas import tpu_sc as plsc

_V = 100000
_D = 128
_B = 4096 * 50
_NC = 2
_NS = 16
_NW = _NC * _NS
_RPW = _B // _NW
_CH = 128
_NCH = _RPW // _CH

_mesh = plsc.VectorSubcoreMesh(
    core_axis_name="c", subcore_axis_name="s", num_cores=_NC, num_subcores=_NS
)


def _dyn_gather(v, idx):
    return lax.gather(
        v, idx[:, None],
        lax.GatherDimensionNumbers(
            offset_dims=(), collapsed_slice_dims=(0,), start_index_map=(0,)),
        (1,),
        mode=lax.GatherScatterMode.PROMISE_IN_BOUNDS)


def _tile_body(x_hbm, w_hbm, sz_hbm, out_hbm, idx_v, w_v, sz_v, out_v,
               sem_w, sem_sz):
    wid = lax.axis_index("s") * _NC + lax.axis_index("c")
    base0 = wid * _RPW
    lane = lax.iota(jnp.int32, 16)
    ge8 = lane // 8
    col_idx = [[64 * h + 4 * lane + j for j in range(4)] for h in range(2)]

    def chunk_body(t, carry):
        base = base0 + t * _CH
        pltpu.sync_copy(x_hbm.at[pl.ds(base, _CH)], idx_v)
        cw = pltpu.async_copy(w_hbm.at[idx_v], w_v, sem_w)
        cs = pltpu.async_copy(sz_hbm.at[idx_v], sz_v, sem_sz)
        cw.wait()
        cs.wait()

        def row_body(r, carry2):
            szr = sz_v[r]
            rbase = jnp.full((16,), r * _D, jnp.int32)
            for h in range(2):
                wv = plsc.bitcast(w_v[r, pl.ds(64 * h, 64)], jnp.int32)
                sp = _dyn_gather(szr, ge8 + 2 * h)
                zp = _dyn_gather(szr, ge8 + 2 * h + 4)
                for j in range(4):
                    b = (wv << (24 - 8 * j)) >> 24
                    f = (b.astype(jnp.float32) - zp) * sp
                    plsc.store_scatter(out_v, [rbase + col_idx[h][j]], f)
            return carry2

        lax.fori_loop(0, _CH, row_body, 0, unroll=2)
        pltpu.sync_copy(out_v, out_hbm.at[pl.ds(base * _D, _CH * _D)])
        return carry

    lax.fori_loop(0, _NCH, chunk_body, 0)


_lookup = pl.kernel(
    _tile_body,
    out_type=jax.ShapeDtypeStruct((_B * _D,), jnp.float32),
    mesh=_mesh,
    scratch_types=[
        pltpu.VMEM((_CH,), jnp.int32),
        pltpu.VMEM((_CH, _D), jnp.int8),
        pltpu.VMEM((_CH, 16), jnp.float32),
        pltpu.VMEM((_CH * _D,), jnp.float32),
        pltpu.SemaphoreType.DMA,
        pltpu.SemaphoreType.DMA,
    ],
    compiler_params=pltpu.CompilerParams(
        needs_layout_passes=False, use_tc_tiling_on_sc=False
    ),
)


@jax.jit
def kernel(x, weight, scale, zero_point):
    n, s = x.shape
    xt = x.T.reshape(-1)
    sz = jnp.concatenate(
        [scale, zero_point.astype(jnp.float32), jnp.zeros((_V, 8), jnp.float32)],
        axis=1,
    )
    out = _lookup(xt, weight, sz)
    return out.reshape(s, n, _D).transpose(1, 0, 2)


# speedup vs baseline: 4.1206x; 1.2589x over previous
import jax
import jax.numpy as jnp
from jax import lax
from jax.experimental import pallas as pl
from jax.experimental.pallas import tpu as pltpu
from jax.experimental.pallas import tpu_sc as plsc

_V = 100000
_D = 128
_B = 4096 * 50
_NC = 2
_NS = 16
_NW = _NC * _NS
_RPW = _B // _NW
_CH = 128
_NCH = _RPW // _CH

_mesh = plsc.VectorSubcoreMesh(
    core_axis_name="c", subcore_axis_name="s", num_cores=_NC, num_subcores=_NS
)


def _dyn_gather(v, idx):
    return lax.gather(
        v, idx[:, None],
        lax.GatherDimensionNumbers(
            offset_dims=(), collapsed_slice_dims=(0,), start_index_map=(0,)),
        (1,),
        mode=lax.GatherScatterMode.PROMISE_IN_BOUNDS)


def _tile_body(x_hbm, w_hbm, sz_hbm, out_hbm,
               idx_v0, idx_v1, w_v0, w_v1, sz_v0, sz_v1, out_v0, out_v1,
               sem_g0, sem_g1, sem_o0, sem_o1):
    idx_v = (idx_v0, idx_v1)
    w_v = (w_v0, w_v1)
    sz_v = (sz_v0, sz_v1)
    out_v = (out_v0, out_v1)
    sem_g = (sem_g0, sem_g1)
    sem_o = (sem_o0, sem_o1)

    wid = lax.axis_index("s") * _NC + lax.axis_index("c")
    base0 = wid * _RPW
    lane = lax.iota(jnp.int32, 16)
    ge8 = lane // 8
    col_idx = [[64 * h + 4 * lane + j for j in range(4)] for h in range(2)]

    def fetch(t, b):
        base = base0 + t * _CH
        pltpu.sync_copy(x_hbm.at[pl.ds(base, _CH)], idx_v[b])
        pltpu.async_copy(w_hbm.at[idx_v[b]], w_v[b], sem_g[b])
        pltpu.async_copy(sz_hbm.at[idx_v[b]], sz_v[b], sem_g[b])

    def wait_gathers(b):
        pltpu.make_async_copy(w_hbm.at[idx_v[b]], w_v[b], sem_g[b]).wait()
        pltpu.make_async_copy(sz_hbm.at[idx_v[b]], sz_v[b], sem_g[b]).wait()

    def drain_out(t_prev, b):
        base = base0 + t_prev * _CH
        pltpu.make_async_copy(
            out_v[b], out_hbm.at[pl.ds(base * _D, _CH * _D)], sem_o[b]
        ).wait()

    def compute_and_send(t, b):
        def row_body(r, carry2):
            szr = sz_v[b][r]
            rbase = jnp.full((16,), r * _D, jnp.int32)
            for h in range(2):
                wv = plsc.bitcast(w_v[b][r, pl.ds(64 * h, 64)], jnp.int32)
                sp = _dyn_gather(szr, ge8 + 2 * h)
                zp = _dyn_gather(szr, ge8 + 2 * h + 4)
                for j in range(4):
                    bb = (wv << (24 - 8 * j)) >> 24
                    f = (bb.astype(jnp.float32) - zp) * sp
                    plsc.store_scatter(out_v[b], [rbase + col_idx[h][j]], f)
            return carry2

        lax.fori_loop(0, _CH, row_body, 0, unroll=2)
        base = base0 + t * _CH
        pltpu.async_copy(out_v[b], out_hbm.at[pl.ds(base * _D, _CH * _D)], sem_o[b])

    fetch(0, 0)
    fetch(1, 1)
    for b in (0, 1):
        wait_gathers(b)
        compute_and_send(b, b)
        fetch(b + 2, b)

    def pair_body(tt, carry):
        for b in (0, 1):
            t = 2 * tt + b
            wait_gathers(b)
            drain_out(t - 2, b)
            compute_and_send(t, b)
            fetch(t + 2, b)
        return carry

    lax.fori_loop(1, _NCH // 2 - 1, pair_body, 0)

    for b in (0, 1):
        t = _NCH - 2 + b
        wait_gathers(b)
        drain_out(t - 2, b)
        compute_and_send(t, b)
    for b in (0, 1):
        drain_out(_NCH - 2 + b, b)


_lookup = pl.kernel(
    _tile_body,
    out_type=jax.ShapeDtypeStruct((_B * _D,), jnp.float32),
    mesh=_mesh,
    scratch_types=[
        pltpu.VMEM((_CH,), jnp.int32),
        pltpu.VMEM((_CH,), jnp.int32),
        pltpu.VMEM((_CH, _D), jnp.int8),
        pltpu.VMEM((_CH, _D), jnp.int8),
        pltpu.VMEM((_CH, 16), jnp.float32),
        pltpu.VMEM((_CH, 16), jnp.float32),
        pltpu.VMEM((_CH * _D,), jnp.float32),
        pltpu.VMEM((_CH * _D,), jnp.float32),
        pltpu.SemaphoreType.DMA,
        pltpu.SemaphoreType.DMA,
        pltpu.SemaphoreType.DMA,
        pltpu.SemaphoreType.DMA,
    ],
    compiler_params=pltpu.CompilerParams(
        needs_layout_passes=False, use_tc_tiling_on_sc=False
    ),
)


@jax.jit
def kernel(x, weight, scale, zero_point):
    n, s = x.shape
    xt = x.T.reshape(-1)
    sz = jnp.concatenate(
        [scale, zero_point.astype(jnp.float32), jnp.zeros((_V, 8), jnp.float32)],
        axis=1,
    )
    out = _lookup(xt, weight, sz)
    return out.reshape(s, n, _D).transpose(1, 0, 2)


# speedup vs baseline: 5.2408x; 1.2719x over previous
import jax
import jax.numpy as jnp
from jax import lax
from jax.experimental import pallas as pl
from jax.experimental.pallas import tpu as pltpu
from jax.experimental.pallas import tpu_sc as plsc

_V = 100000
_D = 128
_B = 4096 * 50
_NC = 2
_NS = 16
_NW = _NC * _NS
_RPW = _B // _NW
_CH = 128
_NCH = _RPW // _CH

_mesh = plsc.VectorSubcoreMesh(
    core_axis_name="c", subcore_axis_name="s", num_cores=_NC, num_subcores=_NS
)


def _dyn_gather(v, idx):
    return lax.gather(
        v, idx[:, None],
        lax.GatherDimensionNumbers(
            offset_dims=(), collapsed_slice_dims=(0,), start_index_map=(0,)),
        (1,),
        mode=lax.GatherScatterMode.PROMISE_IN_BOUNDS)


def _tile_body(x_hbm, w_hbm, sz_hbm, out_hbm,
               idx_v0, idx_v1, w_v0, w_v1, sz_v0, sz_v1, out_v0, out_v1,
               sem_g0, sem_g1, sem_o0, sem_o1):
    idx_v = (idx_v0, idx_v1)
    w_v = (w_v0, w_v1)
    sz_v = (sz_v0, sz_v1)
    out_v = (out_v0, out_v1)
    sem_g = (sem_g0, sem_g1)
    sem_o = (sem_o0, sem_o1)

    wid = lax.axis_index("s") * _NC + lax.axis_index("c")
    base0 = wid * _RPW
    lane = lax.iota(jnp.int32, 16)
    ge8 = lane // 8
    col_idx = [[64 * h + 4 * lane + j for j in range(4)] for h in range(2)]

    def fetch(t, b):
        base = base0 + t * _CH
        pltpu.sync_copy(x_hbm.at[pl.ds(base, _CH)], idx_v[b])
        pltpu.async_copy(w_hbm.at[idx_v[b]], w_v[b], sem_g[b])
        pltpu.async_copy(sz_hbm.at[idx_v[b]], sz_v[b], sem_g[b])

    def wait_gathers(b):
        pltpu.make_async_copy(w_hbm.at[idx_v[b]], w_v[b], sem_g[b]).wait()
        pltpu.make_async_copy(sz_hbm.at[idx_v[b]], sz_v[b], sem_g[b]).wait()

    def drain_out(t_prev, b):
        base = base0 + t_prev * _CH
        pltpu.make_async_copy(
            out_v[b], out_hbm.at[pl.ds(base * _D, _CH * _D)], sem_o[b]
        ).wait()

    def compute_and_send(t, b):
        @plsc.parallel_loop(0, _CH, unroll=4)
        def row_body(r):
            szr = sz_v[b][r]
            rbase = jnp.full((16,), r * _D, jnp.int32)
            for h in range(2):
                wv = plsc.bitcast(w_v[b][r, pl.ds(64 * h, 64)], jnp.int32)
                sp = _dyn_gather(szr, ge8 + 2 * h)
                zp = _dyn_gather(szr, ge8 + 2 * h + 4)
                for j in range(4):
                    bb = (wv << (24 - 8 * j)) >> 24
                    f = (bb.astype(jnp.float32) - zp) * sp
                    plsc.store_scatter(out_v[b], [rbase + col_idx[h][j]], f)
        base = base0 + t * _CH
        pltpu.async_copy(out_v[b], out_hbm.at[pl.ds(base * _D, _CH * _D)], sem_o[b])

    fetch(0, 0)
    fetch(1, 1)
    for b in (0, 1):
        wait_gathers(b)
        compute_and_send(b, b)
        fetch(b + 2, b)

    def pair_body(tt, carry):
        for b in (0, 1):
            t = 2 * tt + b
            wait_gathers(b)
            drain_out(t - 2, b)
            compute_and_send(t, b)
            fetch(t + 2, b)
        return carry

    lax.fori_loop(1, _NCH // 2 - 1, pair_body, 0)

    for b in (0, 1):
        t = _NCH - 2 + b
        wait_gathers(b)
        drain_out(t - 2, b)
        compute_and_send(t, b)
    for b in (0, 1):
        drain_out(_NCH - 2 + b, b)


_lookup = pl.kernel(
    _tile_body,
    out_type=jax.ShapeDtypeStruct((_B * _D,), jnp.float32),
    mesh=_mesh,
    scratch_types=[
        pltpu.VMEM((_CH,), jnp.int32),
        pltpu.VMEM((_CH,), jnp.int32),
        pltpu.VMEM((_CH, _D), jnp.int8),
        pltpu.VMEM((_CH, _D), jnp.int8),
        pltpu.VMEM((_CH, 16), jnp.float32),
        pltpu.VMEM((_CH, 16), jnp.float32),
        pltpu.VMEM((_CH * _D,), jnp.float32),
        pltpu.VMEM((_CH * _D,), jnp.float32),
        pltpu.SemaphoreType.DMA,
        pltpu.SemaphoreType.DMA,
        pltpu.SemaphoreType.DMA,
        pltpu.SemaphoreType.DMA,
    ],
    compiler_params=pltpu.CompilerParams(
        needs_layout_passes=False, use_tc_tiling_on_sc=False
    ),
)


@jax.jit
def kernel(x, weight, scale, zero_point):
    n, s = x.shape
    xt = x.T.reshape(-1)
    sz = jnp.concatenate(
        [scale, zero_point.astype(jnp.float32), jnp.zeros((_V, 8), jnp.float32)],
        axis=1,
    )
    out = _lookup(xt, weight, sz)
    return out.reshape(s, n, _D).transpose(1, 0, 2)
